# Initial kernel scaffold; baseline (speedup 1.0000x reference)
import jax
import jax.numpy as jnp
from jax.experimental import pallas as pl


def kernel(features, mask_proposals, w_head, b_head, w_deconv, b_deconv, w_pred, b_pred):
    raise NotImplementedError("write your pallas kernel here")


# speedup vs baseline: 1.1094x; 1.1094x over previous
import functools

import jax
import jax.numpy as jnp
from jax import lax
from jax.experimental import pallas as pl
from jax.experimental.pallas import tpu as pltpu
from jax.experimental.pallas import tpu_sc as plsc

H, W = 200, 272
C = 256
M = 14
N_ROIS = 512
P = N_ROIS * M * M
HW = H * W
SCALE = 0.25

NW = 32
PPW = P // NW
CHUNK = 56
NCHUNK = PPW // CHUNK

T_STEPS = 25
T_COLS = HW // T_STEPS

NB = 32
MM_ROWS = 1024
MM_STEPS = P // MM_ROWS


def _transpose_body(src_ref, dst_ref):
    dst_ref[...] = src_ref[...].T


def _prep_body(mp_ref, i00_ref, i01_ref, i10_ref, i11_ref, wtb_ref):
    b = mp_ref[...] * SCALE
    x1, y1, x2, y2 = b[:, 0:1], b[:, 1:2], b[:, 2:3], b[:, 3:4]
    bin_w = jnp.maximum(x2 - x1, 1.0) / M
    bin_h = jnp.maximum(y2 - y1, 1.0) / M
    g = lax.broadcasted_iota(jnp.int32, (1, M), 1).astype(jnp.float32) + 0.5
    x = jnp.clip(x1 + g * bin_w, 0.0, W - 1.0)
    y = jnp.clip(y1 + g * bin_h, 0.0, H - 1.0)
    x0f = jnp.floor(x)
    y0f = jnp.floor(y)
    x0 = x0f.astype(jnp.int32)
    y0 = y0f.astype(jnp.int32)
    lx = x - x0f
    ly = y - y0f
    hx = 1.0 - lx
    hy = 1.0 - ly
    row0 = y0 * W
    i00 = row0[:, :, None] + x0[:, None, :]
    i10 = i00 + W
    cap = HW - 1
    i00_ref[...] = i00
    i01_ref[...] = jnp.minimum(i00 + 1, cap)
    i10_ref[...] = jnp.minimum(i10, cap)
    i11_ref[...] = jnp.minimum(i10 + 1, cap)
    w00 = hy[:, :, None] * hx[:, None, :]
    w01 = hy[:, :, None] * lx[:, None, :]
    w10 = ly[:, :, None] * hx[:, None, :]
    w11 = ly[:, :, None] * lx[:, None, :]
    wtb_ref[:, :, :, 0:16] = jnp.broadcast_to(w00[..., None], (NB, M, M, 16))
    wtb_ref[:, :, :, 16:32] = jnp.broadcast_to(w01[..., None], (NB, M, M, 16))
    wtb_ref[:, :, :, 32:48] = jnp.broadcast_to(w10[..., None], (NB, M, M, 16))
    wtb_ref[:, :, :, 48:64] = jnp.broadcast_to(w11[..., None], (NB, M, M, 16))


def _sc_gather_body(table, x00, x01, x10, x11, wtb, out, i0, i1, i2, i3,
                    v0, v1, v2, v3, wv, sem):
    wid = lax.axis_index("s") * 2 + lax.axis_index("c")
    base = wid * PPW

    def chunk(ci, carry):
        p0 = base + ci * CHUNK
        pltpu.sync_copy(x00.at[pl.ds(p0, CHUNK)], i0)
        pltpu.sync_copy(x01.at[pl.ds(p0, CHUNK)], i1)
        pltpu.sync_copy(x10.at[pl.ds(p0, CHUNK)], i2)
        pltpu.sync_copy(x11.at[pl.ds(p0, CHUNK)], i3)
        pltpu.sync_copy(wtb.at[pl.ds(p0, CHUNK)], wv)
        c0 = pltpu.async_copy(table.at[i0], v0, sem)
        c1 = pltpu.async_copy(table.at[i1], v1, sem)
        c2 = pltpu.async_copy(table.at[i2], v2, sem)
        c3 = pltpu.async_copy(table.at[i3], v3, sem)
        c0.wait()
        c1.wait()
        c2.wait()
        c3.wait()

        def point(p, pc):
            w0 = wv[p, pl.ds(0, 16)]
            w1 = wv[p, pl.ds(16, 16)]
            w2 = wv[p, pl.ds(32, 16)]
            w3 = wv[p, pl.ds(48, 16)]
            for k in range(C // 16):
                s = pl.ds(k * 16, 16)
                r = w0 * v0[p, s] + w1 * v1[p, s] + w2 * v2[p, s] + w3 * v3[p, s]
                v0[p, s] = r
            return pc

        lax.fori_loop(0, CHUNK, point, 0)
        pltpu.sync_copy(v0, out.at[pl.ds(p0, CHUNK)])
        return carry

    lax.fori_loop(0, NCHUNK, chunk, 0)


@functools.lru_cache(maxsize=1)
def _sc_gather():
    return pl.kernel(
        _sc_gather_body,
        mesh=plsc.VectorSubcoreMesh(core_axis_name="c", subcore_axis_name="s"),
        out_type=jax.ShapeDtypeStruct((P, C), jnp.float32),
        scratch_types=[
            pltpu.VMEM((CHUNK,), jnp.int32),
            pltpu.VMEM((CHUNK,), jnp.int32),
            pltpu.VMEM((CHUNK,), jnp.int32),
            pltpu.VMEM((CHUNK,), jnp.int32),
            pltpu.VMEM((CHUNK, C), jnp.float32),
            pltpu.VMEM((CHUNK, C), jnp.float32),
            pltpu.VMEM((CHUNK, C), jnp.float32),
            pltpu.VMEM((CHUNK, C), jnp.float32),
            pltpu.VMEM((CHUNK, 64), jnp.float32),
            pltpu.SemaphoreType.DMA,
        ],
    )


def _mm_body(x_ref, wh_ref, bh_ref, w2_ref, b2_ref, wp_ref, bp_ref, o_ref):
    x = x_ref[...]
    h = jnp.maximum(
        jnp.dot(x, wh_ref[...], preferred_element_type=jnp.float32)
        + bh_ref[...], 0.0)
    u = jnp.maximum(
        jnp.dot(h, w2_ref[...], preferred_element_type=jnp.float32)
        + b2_ref[...], 0.0)
    z = jnp.dot(u, wp_ref[...], preferred_element_type=jnp.float32) + bp_ref[...]
    o_ref[...] = jax.nn.sigmoid(z)


def kernel(features, mask_proposals, w_head, b_head, w_deconv, b_deconv,
           w_pred, b_pred):
    f2 = features.reshape(C, HW)
    table = pl.pallas_call(
        _transpose_body,
        grid=(T_STEPS,),
        in_specs=[pl.BlockSpec((C, T_COLS), lambda i: (0, i))],
        out_specs=pl.BlockSpec((T_COLS, C), lambda i: (i, 0)),
        out_shape=jax.ShapeDtypeStruct((HW, C), jnp.float32),
    )(f2)

    ispec = pl.BlockSpec((NB, M, M), lambda i: (i, 0, 0))
    ishape = jax.ShapeDtypeStruct((N_ROIS, M, M), jnp.int32)
    x00, x01, x10, x11, wtb = pl.pallas_call(
        _prep_body,
        grid=(N_ROIS // NB,),
        in_specs=[pl.BlockSpec((NB, 4), lambda i: (i, 0))],
        out_specs=[ispec, ispec, ispec, ispec,
                   pl.BlockSpec((NB, M, M, 64), lambda i: (i, 0, 0, 0))],
        out_shape=[ishape, ishape, ishape, ishape,
                   jax.ShapeDtypeStruct((N_ROIS, M, M, 64), jnp.float32)],
    )(mask_proposals)
    x00 = x00.reshape(P)
    x01 = x01.reshape(P)
    x10 = x10.reshape(P)
    x11 = x11.reshape(P)
    wtb = wtb.reshape(P, 64)

    roi = _sc_gather()(table, x00, x01, x10, x11, wtb)

    whT = jnp.transpose(w_head)
    bh = b_head.reshape(1, 128)
    w2r = w_deconv.reshape(128, 4 * 64)
    b2 = jnp.repeat(b_deconv, 4).reshape(1, 256)
    wp4 = (w_pred[0].reshape(64, 1, 1)
           * jnp.eye(4, dtype=w_pred.dtype).reshape(1, 4, 4)).reshape(256, 4)
    bp = jnp.broadcast_to(b_pred[0:1], (1, 4))

    val = pl.pallas_call(
        _mm_body,
        grid=(MM_STEPS,),
        in_specs=[
            pl.BlockSpec((MM_ROWS, C), lambda i: (i, 0)),
            pl.BlockSpec((C, 128), lambda i: (0, 0)),
            pl.BlockSpec((1, 128), lambda i: (0, 0)),
            pl.BlockSpec((128, 256), lambda i: (0, 0)),
            pl.BlockSpec((1, 256), lambda i: (0, 0)),
            pl.BlockSpec((256, 4), lambda i: (0, 0)),
            pl.BlockSpec((1, 4), lambda i: (0, 0)),
        ],
        out_specs=pl.BlockSpec((MM_ROWS, 4), lambda i: (i, 0)),
        out_shape=jax.ShapeDtypeStruct((P, 4), jnp.float32),
    )(roi, whT, bh, w2r, b2, wp4, bp)

    out = val.reshape(N_ROIS, M, M, 2, 2).transpose(0, 1, 3, 2, 4)
    return out.reshape(N_ROIS, 2 * M, 2 * M)
